# baseline (speedup 1.0000x reference)
import functools

import jax
import jax.numpy as jnp
from jax import lax
from jax.experimental import pallas as pl
from jax.experimental.pallas import tpu as pltpu
from jax.experimental.pallas import tpu_sc as plsc

_BATCH = 32
_SEQ = 128
_D = 14336
_V = 128
_ROWS = _BATCH * _SEQ
_NC = 2
_NS = 16
_D2 = _D // _NC
_NQ = 2
_DQ = _D2 // _NQ
_V4 = _V * 4
_ROWS_PER_S = _ROWS // _NS
_V_PER_S = _V // _NS
_CHUNK = 8
_NGROUP = _ROWS_PER_S // _CHUNK
_NCHUNK = _NGROUP * _NQ
_STREAM_EVERY = 2

_mesh = plsc.VectorSubcoreMesh(core_axis_name="c", subcore_axis_name="s")


@functools.partial(
    pl.kernel,
    mesh=_mesh,
    out_type=jax.ShapeDtypeStruct((_ROWS, _D), jnp.float32),
    scratch_types=[
        pltpu.VMEM((_ROWS_PER_S + 16,), jnp.int32),
        pltpu.VMEM((2 * _ROWS_PER_S,), jnp.int32),
        pltpu.VMEM((2, _CHUNK, _DQ), jnp.float32),
        pltpu.VMEM_SHARED((_V * _D2,), jnp.float32),
        pltpu.SemaphoreType.DMA((2,)),
    ],
)
def _gather(idx_hbm, table_hbm, table4_hbm, out_hbm,
            idx_v, idxq_v, rows_v, table_sh, sems):
    sid = lax.axis_index("s")
    cid = lax.axis_index("c")
    dcol = cid * _D2
    base = sid * _ROWS_PER_S

    for j in range(_V_PER_S):
        r = sid * _V_PER_S + j
        pltpu.sync_copy(
            table_hbm.at[pl.ds(r * _D + dcol, _D2)],
            table_sh.at[pl.ds(r * _D2, _D2)],
        )
    pltpu.sync_copy(idx_hbm.at[pl.ds(base, _ROWS_PER_S)],
                    idx_v.at[pl.ds(0, _ROWS_PER_S)])

    def prep(k, carry):
        vec = idx_v[pl.ds(16 * k, 16)]
        idxq_v[pl.ds(16 * k, 16)] = 4 * vec + 2 * cid
        idxq_v[pl.ds(_ROWS_PER_S + 16 * k, 16)] = 4 * vec + 2 * cid + 1
        return carry

    lax.fori_loop(0, _ROWS_PER_S // 16, prep, 0)
    plsc.subcore_barrier()

    def body(c, carry):
        par = lax.rem(c, 2)
        cc = lax.min(c, _NCHUNK - 1)
        g = lax.div(cc, _NQ)
        q = lax.rem(cc, _NQ)
        buf = rows_v.at[par]
        sem = sems.at[par]
        is_stream = lax.rem(cc, _STREAM_EVERY) == _STREAM_EVERY - 1

        @pl.when(is_stream)
        def _():
            pltpu.async_copy(
                table4_hbm.at[
                    idxq_v.at[pl.ds(_ROWS_PER_S * q + _CHUNK * g, _CHUNK)]
                ],
                buf,
                sem,
            )

        @pl.when(jnp.logical_not(is_stream))
        def _():
            vec = idx_v[pl.ds(8 * g, 16)]
            for j in range(_CHUNK):
                v = vec[j]
                pltpu.async_copy(
                    table_sh.at[pl.ds(v * _D2 + q * _DQ, _DQ)],
                    buf.at[j],
                    sem,
                )

        @pl.when(c > 0)
        def _():
            cp = c - 1
            parp = lax.rem(cp, 2)
            gp = lax.div(cp, _NQ)
            qp = lax.rem(cp, _NQ)
            bufp = rows_v.at[parp]
            semp = sems.at[parp]
            pltpu.make_async_copy(
                out_hbm.at[pl.ds(0, _CHUNK), pl.ds(0, _DQ)], bufp, semp
            ).wait()
            pltpu.sync_copy(
                bufp,
                out_hbm.at[
                    pl.ds(base + gp * _CHUNK, _CHUNK),
                    pl.ds(dcol + qp * _DQ, _DQ),
                ],
            )

        return carry

    lax.fori_loop(0, _NCHUNK + 1, body, 0)

    pltpu.make_async_copy(
        out_hbm.at[pl.ds(0, _CHUNK), pl.ds(0, _DQ)], rows_v.at[0], sems.at[0]
    ).wait()


def kernel(prefix, embedding_table):
    idx = prefix.reshape(_ROWS).astype(jnp.int32)
    table = embedding_table.reshape(_V * _D)
    table4 = embedding_table.reshape(_V4, _DQ)
    out = _gather(idx, table, table4)
    return out.reshape(_BATCH, _SEQ, _D)


# speedup vs baseline: 1.0851x; 1.0851x over previous
import functools

import jax
import jax.numpy as jnp
from jax import lax
from jax.experimental import pallas as pl
from jax.experimental.pallas import tpu as pltpu
from jax.experimental.pallas import tpu_sc as plsc

_BATCH = 32
_SEQ = 128
_D = 14336
_V = 128
_ROWS = _BATCH * _SEQ
_NC = 2
_NS = 16
_D2 = _D // _NC
_NQ = 2
_DQ = _D2 // _NQ
_V4 = _V * 4
_ROWS_PER_S = _ROWS // _NS
_V_PER_S = _V // _NS
_CHUNK = 8
_NGROUP = _ROWS_PER_S // _CHUNK
_NCHUNK = _NGROUP * _NQ
_STREAM_EVERY = 4

_mesh = plsc.VectorSubcoreMesh(core_axis_name="c", subcore_axis_name="s")


@functools.partial(
    pl.kernel,
    mesh=_mesh,
    out_type=jax.ShapeDtypeStruct((_ROWS, _D), jnp.float32),
    scratch_types=[
        pltpu.VMEM((_ROWS_PER_S + 16,), jnp.int32),
        pltpu.VMEM((2 * _ROWS_PER_S,), jnp.int32),
        pltpu.VMEM((2, _CHUNK, _DQ), jnp.float32),
        pltpu.VMEM_SHARED((_V * _D2,), jnp.float32),
        pltpu.SemaphoreType.DMA((2,)),
    ],
)
def _gather(idx_hbm, table_hbm, table4_hbm, out_hbm,
            idx_v, idxq_v, rows_v, table_sh, sems):
    sid = lax.axis_index("s")
    cid = lax.axis_index("c")
    dcol = cid * _D2
    base = sid * _ROWS_PER_S

    for j in range(_V_PER_S):
        r = sid * _V_PER_S + j
        pltpu.sync_copy(
            table_hbm.at[pl.ds(r * _D + dcol, _D2)],
            table_sh.at[pl.ds(r * _D2, _D2)],
        )
    pltpu.sync_copy(idx_hbm.at[pl.ds(base, _ROWS_PER_S)],
                    idx_v.at[pl.ds(0, _ROWS_PER_S)])

    def prep(k, carry):
        vec = idx_v[pl.ds(16 * k, 16)]
        idxq_v[pl.ds(16 * k, 16)] = 4 * vec + 2 * cid
        idxq_v[pl.ds(_ROWS_PER_S + 16 * k, 16)] = 4 * vec + 2 * cid + 1
        return carry

    lax.fori_loop(0, _ROWS_PER_S // 16, prep, 0)
    plsc.subcore_barrier()

    def body(c, carry):
        par = lax.rem(c, 2)
        cc = lax.min(c, _NCHUNK - 1)
        g = lax.div(cc, _NQ)
        q = lax.rem(cc, _NQ)
        buf = rows_v.at[par]
        sem = sems.at[par]
        is_stream = lax.rem(cc, _STREAM_EVERY) == _STREAM_EVERY - 1

        @pl.when(is_stream)
        def _():
            pltpu.async_copy(
                table4_hbm.at[
                    idxq_v.at[pl.ds(_ROWS_PER_S * q + _CHUNK * g, _CHUNK)]
                ],
                buf,
                sem,
            )

        @pl.when(jnp.logical_not(is_stream))
        def _():
            vec = idx_v[pl.ds(8 * g, 16)]
            for j in range(_CHUNK):
                v = vec[j]
                pltpu.async_copy(
                    table_sh.at[pl.ds(v * _D2 + q * _DQ, _DQ)],
                    buf.at[j],
                    sem,
                )

        @pl.when(c > 0)
        def _():
            cp = c - 1
            parp = lax.rem(cp, 2)
            gp = lax.div(cp, _NQ)
            qp = lax.rem(cp, _NQ)
            bufp = rows_v.at[parp]
            semp = sems.at[parp]
            pltpu.make_async_copy(
                out_hbm.at[pl.ds(0, _CHUNK), pl.ds(0, _DQ)], bufp, semp
            ).wait()
            pltpu.sync_copy(
                bufp,
                out_hbm.at[
                    pl.ds(base + gp * _CHUNK, _CHUNK),
                    pl.ds(dcol + qp * _DQ, _DQ),
                ],
            )

        return carry

    lax.fori_loop(0, _NCHUNK + 1, body, 0)

    pltpu.make_async_copy(
        out_hbm.at[pl.ds(0, _CHUNK), pl.ds(0, _DQ)], rows_v.at[0], sems.at[0]
    ).wait()


def kernel(prefix, embedding_table):
    idx = prefix.reshape(_ROWS).astype(jnp.int32)
    table = embedding_table.reshape(_V * _D)
    table4 = embedding_table.reshape(_V4, _DQ)
    out = _gather(idx, table, table4)
    return out.reshape(_BATCH, _SEQ, _D)


# speedup vs baseline: 1.1007x; 1.0143x over previous
import functools

import jax
import jax.numpy as jnp
from jax import lax
from jax.experimental import pallas as pl
from jax.experimental.pallas import tpu as pltpu
from jax.experimental.pallas import tpu_sc as plsc

_BATCH = 32
_SEQ = 128
_D = 14336
_V = 128
_ROWS = _BATCH * _SEQ
_NC = 2
_NS = 16
_D2 = _D // _NC
_NQ = 2
_DQ = _D2 // _NQ
_V4 = _V * 4
_ROWS_PER_S = _ROWS // _NS
_V_PER_S = _V // _NS
_CHUNK = 8
_NGROUP = _ROWS_PER_S // _CHUNK
_NCHUNK = _NGROUP * _NQ
_STREAM_EVERY = 3

_mesh = plsc.VectorSubcoreMesh(core_axis_name="c", subcore_axis_name="s")


@functools.partial(
    pl.kernel,
    mesh=_mesh,
    out_type=jax.ShapeDtypeStruct((_ROWS, _D), jnp.float32),
    scratch_types=[
        pltpu.VMEM((_ROWS_PER_S + 16,), jnp.int32),
        pltpu.VMEM((2 * _ROWS_PER_S,), jnp.int32),
        pltpu.VMEM((2, _CHUNK, _DQ), jnp.float32),
        pltpu.VMEM_SHARED((_V * _D2,), jnp.float32),
        pltpu.SemaphoreType.DMA((2,)),
    ],
)
def _gather(idx_hbm, table_hbm, table4_hbm, out_hbm,
            idx_v, idxq_v, rows_v, table_sh, sems):
    sid = lax.axis_index("s")
    cid = lax.axis_index("c")
    dcol = cid * _D2
    base = sid * _ROWS_PER_S

    for j in range(_V_PER_S):
        r = sid * _V_PER_S + j
        pltpu.sync_copy(
            table_hbm.at[pl.ds(r * _D + dcol, _D2)],
            table_sh.at[pl.ds(r * _D2, _D2)],
        )
    pltpu.sync_copy(idx_hbm.at[pl.ds(base, _ROWS_PER_S)],
                    idx_v.at[pl.ds(0, _ROWS_PER_S)])

    def prep(k, carry):
        vec = idx_v[pl.ds(16 * k, 16)]
        idxq_v[pl.ds(16 * k, 16)] = 4 * vec + 2 * cid
        idxq_v[pl.ds(_ROWS_PER_S + 16 * k, 16)] = 4 * vec + 2 * cid + 1
        return carry

    lax.fori_loop(0, _ROWS_PER_S // 16, prep, 0)
    plsc.subcore_barrier()

    def body(c, carry):
        par = lax.rem(c, 2)
        cc = lax.min(c, _NCHUNK - 1)
        g = lax.div(cc, _NQ)
        q = lax.rem(cc, _NQ)
        buf = rows_v.at[par]
        sem = sems.at[par]
        is_stream = lax.rem(cc, _STREAM_EVERY) == _STREAM_EVERY - 1

        @pl.when(is_stream)
        def _():
            pltpu.async_copy(
                table4_hbm.at[
                    idxq_v.at[pl.ds(_ROWS_PER_S * q + _CHUNK * g, _CHUNK)]
                ],
                buf,
                sem,
            )

        @pl.when(jnp.logical_not(is_stream))
        def _():
            vec = idx_v[pl.ds(8 * g, 16)]
            for j in range(_CHUNK):
                v = vec[j]
                pltpu.async_copy(
                    table_sh.at[pl.ds(v * _D2 + q * _DQ, _DQ)],
                    buf.at[j],
                    sem,
                )

        @pl.when(c > 0)
        def _():
            cp = c - 1
            parp = lax.rem(cp, 2)
            gp = lax.div(cp, _NQ)
            qp = lax.rem(cp, _NQ)
            bufp = rows_v.at[parp]
            semp = sems.at[parp]
            pltpu.make_async_copy(
                out_hbm.at[pl.ds(0, _CHUNK), pl.ds(0, _DQ)], bufp, semp
            ).wait()
            pltpu.sync_copy(
                bufp,
                out_hbm.at[
                    pl.ds(base + gp * _CHUNK, _CHUNK),
                    pl.ds(dcol + qp * _DQ, _DQ),
                ],
            )

        return carry

    lax.fori_loop(0, _NCHUNK + 1, body, 0)

    pltpu.make_async_copy(
        out_hbm.at[pl.ds(0, _CHUNK), pl.ds(0, _DQ)], rows_v.at[0], sems.at[0]
    ).wait()


def kernel(prefix, embedding_table):
    idx = prefix.reshape(_ROWS).astype(jnp.int32)
    table = embedding_table.reshape(_V * _D)
    table4 = embedding_table.reshape(_V4, _DQ)
    out = _gather(idx, table, table4)
    return out.reshape(_BATCH, _SEQ, _D)


# speedup vs baseline: 1.1318x; 1.0283x over previous
import functools

import jax
import jax.numpy as jnp
from jax import lax
from jax.experimental import pallas as pl
from jax.experimental.pallas import tpu as pltpu
from jax.experimental.pallas import tpu_sc as plsc

_BATCH = 32
_SEQ = 128
_D = 14336
_V = 128
_ROWS = _BATCH * _SEQ
_NC = 2
_NS = 16
_D2 = _D // _NC
_NQ = 2
_DQ = _D2 // _NQ
_V4 = _V * 4
_ROWS_PER_S = _ROWS // _NS
_V_PER_S = _V // _NS
_CHUNK = 8
_NGROUP = _ROWS_PER_S // _CHUNK
_NCHUNK = _NGROUP * _NQ
_STREAM_EVERY = 3

_mesh = plsc.VectorSubcoreMesh(core_axis_name="c", subcore_axis_name="s")


@functools.partial(
    pl.kernel,
    mesh=_mesh,
    out_type=jax.ShapeDtypeStruct((_ROWS, _D), jnp.float32),
    scratch_types=[
        pltpu.VMEM((_ROWS_PER_S + 16,), jnp.int32),
        pltpu.VMEM((2 * _ROWS_PER_S,), jnp.int32),
        pltpu.VMEM((2, _CHUNK, _DQ), jnp.float32),
        pltpu.VMEM_SHARED((_V * _D2,), jnp.float32),
        pltpu.SemaphoreType.DMA((2,)),
    ],
)
def _gather(idx_hbm, table_hbm, table4_hbm, out_hbm,
            idx_v, idxq_v, rows_v, table_sh, sems):
    sid = lax.axis_index("s")
    cid = lax.axis_index("c")
    dcol = cid * _D2
    base = sid * _ROWS_PER_S

    for j in range(_V_PER_S):
        r = sid * _V_PER_S + j
        pltpu.async_copy(
            table_hbm.at[pl.ds(r * _D + dcol, _D2)],
            table_sh.at[pl.ds(r * _D2, _D2)],
            sems.at[0],
        )
    pltpu.sync_copy(idx_hbm.at[pl.ds(base, _ROWS_PER_S)],
                    idx_v.at[pl.ds(0, _ROWS_PER_S)])
    pltpu.make_async_copy(
        table_hbm.at[pl.ds(0, _V_PER_S * _D2)],
        table_sh.at[pl.ds(sid * _V_PER_S * _D2, _V_PER_S * _D2)],
        sems.at[0],
    ).wait()

    def prep(k, carry):
        vec = idx_v[pl.ds(16 * k, 16)]
        idxq_v[pl.ds(16 * k, 16)] = 4 * vec + 2 * cid
        idxq_v[pl.ds(_ROWS_PER_S + 16 * k, 16)] = 4 * vec + 2 * cid + 1
        return carry

    lax.fori_loop(0, _ROWS_PER_S // 16, prep, 0)
    plsc.subcore_barrier()

    def body(c, carry):
        par = lax.rem(c, 2)
        cc = lax.min(c, _NCHUNK - 1)
        g = lax.div(cc, _NQ)
        q = lax.rem(cc, _NQ)
        buf = rows_v.at[par]
        sem = sems.at[par]
        live = c < _NCHUNK
        is_stream = lax.rem(cc, _STREAM_EVERY) == _STREAM_EVERY - 1

        @pl.when(jnp.logical_and(live, is_stream))
        def _():
            pltpu.async_copy(
                table4_hbm.at[
                    idxq_v.at[pl.ds(_ROWS_PER_S * q + _CHUNK * g, _CHUNK)]
                ],
                buf,
                sem,
            )

        @pl.when(jnp.logical_and(live, jnp.logical_not(is_stream)))
        def _():
            vec = idx_v[pl.ds(8 * g, 16)]
            for j in range(_CHUNK):
                v = vec[j]
                pltpu.async_copy(
                    table_sh.at[pl.ds(v * _D2 + q * _DQ, _DQ)],
                    buf.at[j],
                    sem,
                )

        @pl.when(c > 0)
        def _():
            cp = c - 1
            parp = lax.rem(cp, 2)
            gp = lax.div(cp, _NQ)
            qp = lax.rem(cp, _NQ)
            bufp = rows_v.at[parp]
            semp = sems.at[parp]
            pltpu.make_async_copy(
                out_hbm.at[pl.ds(0, _CHUNK), pl.ds(0, _DQ)], bufp, semp
            ).wait()
            pltpu.sync_copy(
                bufp,
                out_hbm.at[
                    pl.ds(base + gp * _CHUNK, _CHUNK),
                    pl.ds(dcol + qp * _DQ, _DQ),
                ],
            )

        return carry

    lax.fori_loop(0, _NCHUNK + 1, body, 0)


def kernel(prefix, embedding_table):
    idx = prefix.reshape(_ROWS).astype(jnp.int32)
    table = embedding_table.reshape(_V * _D)
    table4 = embedding_table.reshape(_V4, _DQ)
    out = _gather(idx, table, table4)
    return out.reshape(_BATCH, _SEQ, _D)
